# Initial kernel scaffold; baseline (speedup 1.0000x reference)
import jax
import jax.numpy as jnp
from jax.experimental import pallas as pl


def kernel(logits, pred_boxes, boxes, class_labels):
    raise NotImplementedError("write your pallas kernel here")


# speedup vs baseline: 169.1971x; 169.1971x over previous
import jax
import jax.numpy as jnp
from jax import lax
from jax.experimental import pallas as pl
from jax.experimental.pallas import tpu as pltpu

_NQ = 1000
_NT = 64
_NC = 91


def _matcher_kernel(labels_ref, logits_ref, pred_ref, tbox_ref,
                    rows_ref, cols_ref, cls_ref, w_ref):
    def gath(t, carry):
        lbl = labels_ref[0, t]
        cls_ref[pl.ds(t, 1), :] = logits_ref[0, pl.ds(lbl, 1), :]
        return carry

    lax.fori_loop(0, _NT, gath, 0)
    cost_class = -jax.nn.sigmoid(cls_ref[...])

    pb = pred_ref[0]
    q_cx, q_cy = pb[0:1, :], pb[1:2, :]
    q_w, q_h = pb[2:3, :], pb[3:4, :]
    tb = tbox_ref[...]
    t_cx, t_cy = tb[:, 0:1], tb[:, 1:2]
    t_w, t_h = tb[:, 2:3], tb[:, 3:4]

    cost_bbox = ((jnp.abs(q_cx - t_cx) + jnp.abs(q_cy - t_cy))
                 + jnp.abs(q_w - t_w)) + jnp.abs(q_h - t_h)

    qx0, qy0 = q_cx - 0.5 * q_w, q_cy - 0.5 * q_h
    qx1, qy1 = q_cx + 0.5 * q_w, q_cy + 0.5 * q_h
    tx0, ty0 = t_cx - 0.5 * t_w, t_cy - 0.5 * t_h
    tx1, ty1 = t_cx + 0.5 * t_w, t_cy + 0.5 * t_h

    area_q = (qx1 - qx0) * (qy1 - qy0)
    area_t = (tx1 - tx0) * (ty1 - ty0)
    wx = jnp.maximum(jnp.minimum(qx1, tx1) - jnp.maximum(qx0, tx0), 0.0)
    wy = jnp.maximum(jnp.minimum(qy1, ty1) - jnp.maximum(qy0, ty0), 0.0)
    inter = wx * wy
    union = area_q + area_t - inter
    iou = inter / union
    ex = jnp.maximum(qx1, tx1) - jnp.minimum(qx0, tx0)
    ey = jnp.maximum(qy1, ty1) - jnp.minimum(qy0, ty0)
    area_e = jnp.maximum(ex, 0.0) * jnp.maximum(ey, 0.0)
    cost_giou = -(iou - (area_e - union) / area_e)

    w_ref[...] = (cost_bbox + cost_class) + cost_giou

    riota = lax.broadcasted_iota(jnp.int32, (_NT, _NQ), 1)
    ciota = lax.broadcasted_iota(jnp.int32, (_NT, _NQ), 0)
    flat = riota * _NT + ciota
    iota64 = lax.broadcasted_iota(jnp.int32, (1, _NT), 1)
    inf = jnp.float32(jnp.inf)

    def step(i, carry):
        rows_v, cols_v = carry
        w = w_ref[...]
        m = jnp.min(w)
        f = jnp.min(jnp.where(w == m, flat, jnp.int32(2147483647)))
        r = f // _NT
        c = f - r * _NT
        rows_v = jnp.where(iota64 == i, r, rows_v)
        cols_v = jnp.where(iota64 == i, c, cols_v)
        w_ref[...] = jnp.where((riota == r) | (ciota == c), inf, w)
        return rows_v, cols_v

    zeros = jnp.zeros((1, _NT), jnp.int32)
    rows_v, cols_v = lax.fori_loop(0, _NT, step, (zeros, zeros))
    rows_ref[0] = rows_v
    cols_ref[0] = cols_v


def kernel(logits, pred_boxes, boxes, class_labels):
    bs = logits.shape[0]
    logits_t = jnp.swapaxes(logits, 1, 2)
    pred_t = jnp.swapaxes(pred_boxes, 1, 2)
    tbox = boxes[0]
    labels = class_labels[0].reshape(1, _NT)

    rows, cols = pl.pallas_call(
        _matcher_kernel,
        grid=(bs,),
        in_specs=[
            pl.BlockSpec(memory_space=pltpu.SMEM),
            pl.BlockSpec((1, _NC, _NQ), lambda b: (b, 0, 0)),
            pl.BlockSpec((1, 4, _NQ), lambda b: (b, 0, 0)),
            pl.BlockSpec((_NT, 4), lambda b: (0, 0)),
        ],
        out_specs=[
            pl.BlockSpec((1, 1, _NT), lambda b: (b, 0, 0)),
            pl.BlockSpec((1, 1, _NT), lambda b: (b, 0, 0)),
        ],
        out_shape=[jax.ShapeDtypeStruct((bs, 1, _NT), jnp.int32)] * 2,
        scratch_shapes=[
            pltpu.VMEM((_NT, _NQ), jnp.float32),
            pltpu.VMEM((_NT, _NQ), jnp.float32),
        ],
    )(labels, logits_t, pred_t, tbox)
    return rows.reshape(bs, _NT), cols.reshape(bs, _NT)


# speedup vs baseline: 667.9195x; 3.9476x over previous
import jax
import jax.numpy as jnp
from jax import lax
from jax.experimental import pallas as pl
from jax.experimental.pallas import tpu as pltpu
from jax.experimental.pallas import tpu_sc as plsc

_NQ = 1000
_NQP = 1024
_NT = 64
_NC = 91
_BIG = 2 ** 30


def _cost_kernel(labels_ref, logits_ref, pred_ref, tbox_ref,
                 cost_ref, cv_ref, cr_ref, cls_ref):
    def gath(t, carry):
        lbl = labels_ref[0, t]
        cls_ref[pl.ds(t, 1), :] = logits_ref[0, pl.ds(lbl, 1), :]
        return carry

    lax.fori_loop(0, _NT, gath, 0)
    cost_class = -jax.nn.sigmoid(cls_ref[...])

    pb = pred_ref[0]
    q_cx, q_cy = pb[0:1, :], pb[1:2, :]
    q_w, q_h = pb[2:3, :], pb[3:4, :]
    tb = tbox_ref[...]
    t_cx, t_cy = tb[:, 0:1], tb[:, 1:2]
    t_w, t_h = tb[:, 2:3], tb[:, 3:4]

    cost_bbox = ((jnp.abs(q_cx - t_cx) + jnp.abs(q_cy - t_cy))
                 + jnp.abs(q_w - t_w)) + jnp.abs(q_h - t_h)

    qx0, qy0 = q_cx - 0.5 * q_w, q_cy - 0.5 * q_h
    qx1, qy1 = q_cx + 0.5 * q_w, q_cy + 0.5 * q_h
    tx0, ty0 = t_cx - 0.5 * t_w, t_cy - 0.5 * t_h
    tx1, ty1 = t_cx + 0.5 * t_w, t_cy + 0.5 * t_h

    area_q = (qx1 - qx0) * (qy1 - qy0)
    area_t = (tx1 - tx0) * (ty1 - ty0)
    wx = jnp.maximum(jnp.minimum(qx1, tx1) - jnp.maximum(qx0, tx0), 0.0)
    wy = jnp.maximum(jnp.minimum(qy1, ty1) - jnp.maximum(qy0, ty0), 0.0)
    inter = wx * wy
    union = area_q + area_t - inter
    iou = inter / union
    ex = jnp.maximum(qx1, tx1) - jnp.minimum(qx0, tx0)
    ey = jnp.maximum(qy1, ty1) - jnp.minimum(qy0, ty0)
    area_e = jnp.maximum(ex, 0.0) * jnp.maximum(ey, 0.0)
    cost_giou = -(iou - (area_e - union) / area_e)

    cm = (cost_bbox + cost_class) + cost_giou
    riota = lax.broadcasted_iota(jnp.int32, (_NT, _NQP), 1)
    cm = jnp.where(riota >= _NQ, jnp.float32(jnp.inf), cm)
    cost_ref[0] = cm
    m = jnp.min(cm, axis=1, keepdims=True)
    cv_ref[0] = m
    cr_ref[0] = jnp.min(jnp.where(cm == m, riota, _BIG),
                        axis=1, keepdims=True)


def _sc_greedy(cost_hbm, cv_hbm, cr_hbm, rows_hbm, cols_hbm,
               w_v, cv_v, cr_v, rows_v, cols_v):
    b = lax.axis_index("s") * 2 + lax.axis_index("c")
    nb = cost_hbm.shape[0]
    iota16 = lax.iota(jnp.int32, 16)
    z16 = jnp.zeros((16,), jnp.int32)
    lane0 = iota16 == 0
    inf = jnp.float32(jnp.inf)

    def spl_i(x):
        return jnp.full((16,), x, jnp.int32)

    def spl_f(x):
        return jnp.full((16,), x, jnp.float32)

    def scal(v):
        return jnp.min(v) if v.ndim else v

    def col_min(col):
        def body(k, carry):
            lmin, lrow = carry
            rows = k * 16 + iota16
            val = plsc.load_gather(w_v, [spl_i(col), rows])
            lt = val < lmin
            lmin = jnp.where(lt, val, lmin)
            lrow = jnp.where(lt, rows, lrow)
            return lmin, lrow

        lmin, lrow = lax.fori_loop(0, _NQP // 16, body,
                                   (spl_f(inf), spl_i(_BIG)))
        m2 = scal(lmin)
        r2 = scal(jnp.min(jnp.where(lmin == m2, lrow, _BIG)))
        return m2, r2

    @pl.when(b < nb)
    def _():
        pltpu.sync_copy(cost_hbm.at[b], w_v)
        pltpu.sync_copy(cv_hbm.at[b], cv_v)
        pltpu.sync_copy(cr_hbm.at[b], cr_v)

        def step(i, carry):
            vm = spl_f(inf)
            chunks = []
            for c4 in range(4):
                ci = c4 * 16 + iota16
                vals = plsc.load_gather(cv_v, [ci])
                rows = plsc.load_gather(cr_v, [ci])
                chunks.append((ci, vals, rows))
                vm = jnp.minimum(vm, vals)
            m = scal(vm)
            fm = spl_i(_BIG)
            for ci, vals, rows in chunks:
                fm = jnp.minimum(
                    fm, jnp.where(vals == m, rows * _NT + ci, _BIG))
            f = scal(fm)
            r = f // _NT
            c = f - r * _NT

            plsc.store_scatter(rows_v, [spl_i(i)], spl_i(r), mask=lane0)
            plsc.store_scatter(cols_v, [spl_i(i)], spl_i(c), mask=lane0)
            plsc.store_scatter(cv_v, [spl_i(c)], spl_f(inf), mask=lane0)
            for c4 in range(4):
                ci = c4 * 16 + iota16
                plsc.store_scatter(w_v, [ci, spl_i(r)], spl_f(inf))
            for c4 in range(4):
                ci = c4 * 16 + iota16
                vals = plsc.load_gather(cv_v, [ci])
                rows = plsc.load_gather(cr_v, [ci])
                stale = (rows == r) & (vals < inf)

                def scond(mask):
                    return scal(-mask.astype(jnp.int32)) < 0

                def sbody(mask):
                    j = scal(plsc.all_reduce_ffs(mask))
                    col = c4 * 16 + j
                    m2, r2 = col_min(col)
                    plsc.store_scatter(cv_v, [spl_i(col)], spl_f(m2),
                                       mask=lane0)
                    plsc.store_scatter(cr_v, [spl_i(col)], spl_i(r2),
                                       mask=lane0)
                    return mask & (iota16 != j)

                lax.while_loop(scond, sbody, stale)
            return carry

        lax.fori_loop(0, _NT, step, 0)
        pltpu.sync_copy(rows_v, rows_hbm.at[b])
        pltpu.sync_copy(cols_v, cols_hbm.at[b])


def kernel(logits, pred_boxes, boxes, class_labels):
    bs = logits.shape[0]
    logits_t = jnp.pad(jnp.swapaxes(logits, 1, 2),
                       ((0, 0), (0, 0), (0, _NQP - _NQ)))
    pred_t = jnp.pad(jnp.swapaxes(pred_boxes, 1, 2),
                     ((0, 0), (0, 0), (0, _NQP - _NQ)))
    tbox = boxes[0]
    labels = class_labels[0].reshape(1, _NT)

    cost, cv, cr = pl.pallas_call(
        _cost_kernel,
        grid=(bs,),
        in_specs=[
            pl.BlockSpec(memory_space=pltpu.SMEM),
            pl.BlockSpec((1, _NC, _NQP), lambda b: (b, 0, 0)),
            pl.BlockSpec((1, 4, _NQP), lambda b: (b, 0, 0)),
            pl.BlockSpec((_NT, 4), lambda b: (0, 0)),
        ],
        out_specs=[
            pl.BlockSpec((1, _NT, _NQP), lambda b: (b, 0, 0)),
            pl.BlockSpec((1, _NT, 1), lambda b: (b, 0, 0)),
            pl.BlockSpec((1, _NT, 1), lambda b: (b, 0, 0)),
        ],
        out_shape=[
            jax.ShapeDtypeStruct((bs, _NT, _NQP), jnp.float32),
            jax.ShapeDtypeStruct((bs, _NT, 1), jnp.float32),
            jax.ShapeDtypeStruct((bs, _NT, 1), jnp.int32),
        ],
        scratch_shapes=[pltpu.VMEM((_NT, _NQP), jnp.float32)],
    )(labels, logits_t, pred_t, tbox)

    mesh = plsc.VectorSubcoreMesh(core_axis_name="c", subcore_axis_name="s")
    rows, cols = pl.kernel(
        _sc_greedy,
        out_type=[jax.ShapeDtypeStruct((bs, _NT), jnp.int32)] * 2,
        mesh=mesh,
        scratch_types=[
            pltpu.VMEM((_NT, _NQP), jnp.float32),
            pltpu.VMEM((_NT,), jnp.float32),
            pltpu.VMEM((_NT,), jnp.int32),
            pltpu.VMEM((_NT,), jnp.int32),
            pltpu.VMEM((_NT,), jnp.int32),
        ],
        compiler_params=pltpu.CompilerParams(needs_layout_passes=False),
    )(cost, cv.reshape(bs, _NT), cr.reshape(bs, _NT))
    return rows, cols
